# baseline (speedup 1.0000x reference)
import functools

import jax
import jax.numpy as jnp
from jax import lax
from jax.experimental import pallas as pl
from jax.experimental.pallas import tpu as pltpu
from jax.experimental.pallas import tpu_sc as plsc

V = 100000
D = 64
NC, NS = 2, 16
NW = NC * NS
L = 16


def _sc_gather_t(table_t, idxt):
    d_, v_ = table_t.shape
    h_, nbt, bl = idxt.shape
    npass = d_ // NW
    mesh = plsc.VectorSubcoreMesh(core_axis_name="c", subcore_axis_name="s")

    @functools.partial(
        pl.kernel,
        mesh=mesh,
        compiler_params=pltpu.CompilerParams(
            use_tc_tiling_on_sc=False, needs_layout_passes=False
        ),
        out_type=jax.ShapeDtypeStruct((h_, 8, nbt, 8, bl), jnp.float32),
        scratch_types=[
            pltpu.VMEM((v_,), jnp.float32),
            pltpu.VMEM((2, nbt, bl), jnp.int32),
            pltpu.VMEM((2, nbt, 1, bl), jnp.float32),
            pltpu.SemaphoreType.DMA,
            pltpu.SemaphoreType.DMA,
            pltpu.SemaphoreType.DMA,
            pltpu.SemaphoreType.DMA,
            pltpu.SemaphoreType.DMA,
        ],
    )
    def k(tab_hbm, idx_hbm, out_hbm, row_v, idx_v, stg_v, rsem, i0, i1, o0, o1):
        isems = (i0, i1)
        osems = (o0, o1)
        wid = lax.axis_index("s") * NC + lax.axis_index("c")

        def istart(h, s):
            pltpu.async_copy(idx_hbm.at[h], idx_v.at[s], isems[s])

        def iwait(h, s):
            pltpu.make_async_copy(idx_hbm.at[h], idx_v.at[s], isems[s]).wait()

        def odst(h, dt, di):
            return out_hbm.at[h, dt, slice(None), pl.ds(di, 1), slice(None)]

        def ostart(h, dt, di, s):
            pltpu.async_copy(stg_v.at[s], odst(h, dt, di), osems[s])

        def owait(h, dt, di, s):
            pltpu.make_async_copy(stg_v.at[s], odst(h, dt, di), osems[s]).wait()

        dtp = dip = None
        for p in range(npass):
            d = p * NW + wid
            dt = d // 8
            di = lax.rem(d, 8)
            pltpu.async_copy(tab_hbm.at[d], row_v, rsem)
            if p > 0:
                owait(h_ - 2, dtp, dip, 0)
                owait(h_ - 1, dtp, dip, 1)
            pltpu.make_async_copy(tab_hbm.at[d], row_v, rsem).wait()
            istart(0, 0)
            istart(1, 1)

            def g_body(g, carry):
                for s in range(2):
                    h = 2 * g + s
                    iwait(h, s)

                    @pl.when(g >= 1)
                    def _():
                        owait(h - 2, dt, di, s)

                    def bt_body(bt, c):
                        for l in range(8):
                            vidx = idx_v[s, bt, pl.ds(l * L, L)]
                            vals = plsc.load_gather(row_v, [vidx])
                            stg_v[s, bt, 0, pl.ds(l * L, L)] = vals
                        return c

                    lax.fori_loop(0, nbt, bt_body, 0)
                    ostart(h, dt, di, s)

                    @pl.when(g < h_ // 2 - 1)
                    def _():
                        istart(h + 2, s)
                return carry

            lax.fori_loop(0, h_ // 2, g_body, 0)
            dtp, dip = dt, di

        owait(h_ - 2, dtp, dip, 0)
        owait(h_ - 1, dtp, dip, 1)

    return k(table_t, idxt)


def kernel(variate_ids, variate_embed_weight):
    b, h = variate_ids.shape
    table_t = variate_embed_weight.T
    idxt = variate_ids.T.reshape(h, b // 128, 128).astype(jnp.int32)
    out5 = _sc_gather_t(table_t, idxt)
    return out5.transpose(2, 4, 0, 1, 3).reshape(b, h, D)


# speedup vs baseline: 1.0131x; 1.0131x over previous
import functools

import jax
import jax.numpy as jnp
from jax import lax
from jax.experimental import pallas as pl
from jax.experimental.pallas import tpu as pltpu
from jax.experimental.pallas import tpu_sc as plsc

V = 100000
D = 64
NC, NS = 2, 16
NW = NC * NS
L = 16


def _sc_gather_t(table_t, idxt):
    d_, v_ = table_t.shape
    h_, nbt, bl = idxt.shape
    npass = d_ // NW
    mesh = plsc.VectorSubcoreMesh(core_axis_name="c", subcore_axis_name="s")

    @functools.partial(
        pl.kernel,
        mesh=mesh,
        compiler_params=pltpu.CompilerParams(
            use_tc_tiling_on_sc=False, needs_layout_passes=False
        ),
        out_type=jax.ShapeDtypeStruct((h_, 8, nbt, 8, bl), jnp.float32),
        scratch_types=[
            pltpu.VMEM((v_,), jnp.float32),
            pltpu.VMEM((2, nbt, bl), jnp.int32),
            pltpu.VMEM((2, nbt, 1, bl), jnp.float32),
            pltpu.SemaphoreType.DMA,
            pltpu.SemaphoreType.DMA,
            pltpu.SemaphoreType.DMA,
            pltpu.SemaphoreType.DMA,
            pltpu.SemaphoreType.DMA,
        ],
    )
    def k(tab_hbm, idx_hbm, out_hbm, row_v, idx_v, stg_v, rsem, i0, i1, o0, o1):
        isems = (i0, i1)
        osems = (o0, o1)
        wid = lax.axis_index("s") * NC + lax.axis_index("c")

        def istart(h, s):
            pltpu.async_copy(idx_hbm.at[h], idx_v.at[s], isems[s])

        def iwait(h, s):
            pltpu.make_async_copy(idx_hbm.at[h], idx_v.at[s], isems[s]).wait()

        def odst(h, dt, di):
            return out_hbm.at[h, dt, slice(None), pl.ds(di, 1), slice(None)]

        def ostart(h, dt, di, s):
            pltpu.async_copy(stg_v.at[s], odst(h, dt, di), osems[s])

        def owait(h, dt, di, s):
            pltpu.make_async_copy(stg_v.at[s], odst(h, dt, di), osems[s]).wait()

        dtp = dip = None
        for p in range(npass):
            d = p * NW + wid
            dt = d // 8
            di = lax.rem(d, 8)
            pltpu.async_copy(tab_hbm.at[d], row_v, rsem)
            if p > 0:
                owait(h_ - 2, dtp, dip, 0)
                owait(h_ - 1, dtp, dip, 1)
            pltpu.make_async_copy(tab_hbm.at[d], row_v, rsem).wait()
            istart(0, 0)
            istart(1, 1)

            def g_body(g, carry):
                for s in range(2):
                    h = 2 * g + s
                    iwait(h, s)

                    @pl.when(g >= 1)
                    def _():
                        owait(h - 2, dt, di, s)

                    def bt_body(bq, c):
                        for u in range(4):
                            bt = bq * 4 + u
                            for l in range(8):
                                vidx = idx_v[s, bt, pl.ds(l * L, L)]
                                vals = plsc.load_gather(row_v, [vidx])
                                stg_v[s, bt, 0, pl.ds(l * L, L)] = vals
                        return c

                    lax.fori_loop(0, nbt // 4, bt_body, 0)
                    ostart(h, dt, di, s)

                    @pl.when(g < h_ // 2 - 1)
                    def _():
                        istart(h + 2, s)
                return carry

            lax.fori_loop(0, h_ // 2, g_body, 0)
            dtp, dip = dt, di

        owait(h_ - 2, dtp, dip, 0)
        owait(h_ - 1, dtp, dip, 1)

    return k(table_t, idxt)


def kernel(variate_ids, variate_embed_weight):
    b, h = variate_ids.shape
    table_t = variate_embed_weight.T
    idxt = variate_ids.T.reshape(h, b // 128, 128).astype(jnp.int32)
    out5 = _sc_gather_t(table_t, idxt)
    return out5.transpose(2, 4, 0, 1, 3).reshape(b, h, D)


# speedup vs baseline: 1.3272x; 1.3101x over previous
import functools

import jax
import jax.numpy as jnp
from jax import lax
from jax.experimental import pallas as pl
from jax.experimental.pallas import tpu as pltpu
from jax.experimental.pallas import tpu_sc as plsc

D = 64
NC, NS = 2, 16
NW = NC * NS
NBUF = 4


def _sc_gather(table, idxn, b_, h_):
    nblk, blk = idxn.shape
    nbt = b_ // 128
    mblk = nblk // NW
    ring = 2 * NBUF
    nch = mblk * 8
    ngrp = nch // ring
    mesh = plsc.VectorSubcoreMesh(core_axis_name="c", subcore_axis_name="s")

    @functools.partial(
        pl.kernel,
        mesh=mesh,
        compiler_params=pltpu.CompilerParams(use_tc_tiling_on_sc=False),
        out_type=jax.ShapeDtypeStruct((b_, h_ * 128), jnp.float32),
        scratch_types=[
            pltpu.VMEM((mblk, blk), jnp.int32),
            pltpu.VMEM((ring, 128, D), jnp.float32),
        ] + [pltpu.SemaphoreType.DMA] * (2 * ring),
    )
    def k(table_hbm, idx_hbm, out_hbm, idx_v, rows_v, *sems):
        gsems, osems = sems[:ring], sems[ring:]
        wid = lax.axis_index("s") * NC + lax.axis_index("c")
        pltpu.sync_copy(idx_hbm.at[pl.ds(wid * mblk, mblk)], idx_v)

        def odst(j):
            m = wid * mblk + j // 8
            ht = m // nbt
            bt = lax.rem(m, nbt)
            h = ht * 8 + lax.rem(j, 8)
            return out_hbm.at[pl.ds(bt * 128, 128), pl.ds(h * 128, D)]

        def gstart(j, b):
            pltpu.async_copy(
                table_hbm.at[idx_v.at[j // 8, pl.ds(lax.rem(j, 8) * 128, 128)]],
                rows_v.at[b],
                gsems[b],
            )

        def gwait(j, b):
            pltpu.make_async_copy(
                table_hbm.at[idx_v.at[j // 8, pl.ds(lax.rem(j, 8) * 128, 128)]],
                rows_v.at[b],
                gsems[b],
            ).wait()

        def ostart(j, b):
            pltpu.async_copy(rows_v.at[b], odst(j), osems[b])

        def owait(j, b):
            pltpu.make_async_copy(rows_v.at[b], odst(j), osems[b]).wait()

        for b in range(NBUF):
            gstart(b, b)

        def body(g, carry):
            for b in range(ring):
                j = g * ring + b
                gwait(j, b)
                ostart(j, b)
                bn = (b + NBUF) % ring
                jn = j + NBUF
                if b < NBUF:
                    @pl.when(g >= 1)
                    def _():
                        owait(jn - ring, bn)
                        gstart(jn, bn)

                    @pl.when(g < 1)
                    def _():
                        gstart(jn, bn)
                else:
                    @pl.when(g < ngrp - 1)
                    def _():
                        owait(jn - ring, bn)
                        gstart(jn, bn)
            return carry

        lax.fori_loop(0, ngrp, body, 0)

        for b in range(ring):
            owait(nch - ring + b, b)

    return k(table, idxn)


def kernel(variate_ids, variate_embed_weight):
    b, h = variate_ids.shape
    idxn = (
        variate_ids.T.reshape(h // 8, 8, b // 128, 128)
        .transpose(0, 2, 1, 3)
        .reshape((h // 8) * (b // 128), 8 * 128)
        .astype(jnp.int32)
    )
    out = _sc_gather(variate_embed_weight, idxn, b, h)
    return out.reshape(b, h, 128)[:, :, :D]
